# Initial kernel scaffold; baseline (speedup 1.0000x reference)
import jax
import jax.numpy as jnp
from jax.experimental import pallas as pl


def kernel(x, graph, W1, Wq1, Wk1, b1, W2, Wq2, Wk2, b2):
    raise NotImplementedError("write your pallas kernel here")


# speedup vs baseline: 2.5880x; 2.5880x over previous
import math

import jax
import jax.numpy as jnp
import numpy as np
from jax.experimental import pallas as pl
from jax.experimental.pallas import tpu as pltpu

_N = 8192
_D = 64
_A = 32
_RB = 256
_PB = 1024

_INV_SQRT_A = np.float32(1.0 / math.sqrt(_A))
_NEG = np.float32(-1e9)
_SLOPE = np.float32(0.2)


def _proj_kernel(x_ref, W_ref, Wq_ref, Wk_ref, b_ref, h_ref, q_ref, kT_ref):
    x = x_ref[...]
    h_ref[...] = (
        jnp.dot(x, W_ref[...], preferred_element_type=jnp.float32) + b_ref[...]
    )
    q_ref[...] = (
        jnp.dot(x, Wq_ref[...], preferred_element_type=jnp.float32) * _INV_SQRT_A
    )
    kT_ref[...] = jax.lax.dot_general(
        Wk_ref[...], x, (((0,), (1,)), ((), ())),
        preferred_element_type=jnp.float32,
    )


def _project(x, W, Wq, Wk, b):
    grid = (_N // _PB,)
    return pl.pallas_call(
        _proj_kernel,
        grid=grid,
        in_specs=[
            pl.BlockSpec((_PB, _D), lambda i: (i, 0)),
            pl.BlockSpec((_D, _D), lambda i: (0, 0)),
            pl.BlockSpec((_D, _A), lambda i: (0, 0)),
            pl.BlockSpec((_D, _A), lambda i: (0, 0)),
            pl.BlockSpec((1, _D), lambda i: (0, 0)),
        ],
        out_specs=[
            pl.BlockSpec((_PB, _D), lambda i: (i, 0)),
            pl.BlockSpec((_PB, _A), lambda i: (i, 0)),
            pl.BlockSpec((_A, _PB), lambda i: (0, i)),
        ],
        out_shape=[
            jax.ShapeDtypeStruct((_N, _D), jnp.float32),
            jax.ShapeDtypeStruct((_N, _A), jnp.float32),
            jax.ShapeDtypeStruct((_A, _N), jnp.float32),
        ],
    )(x, W, Wq, Wk, b.reshape(1, _D))


def _att_kernel(x_ref, g_ref, q_ref, kT_ref, h_ref, C_ref, o_ref, *, relu):
    i = pl.program_id(0)
    e = jnp.dot(q_ref[...], kT_ref[...], preferred_element_type=jnp.float32)
    e = jnp.where(e >= 0, e, e * _SLOPE)
    rows = i * _RB + jax.lax.broadcasted_iota(jnp.int32, (_RB, _N), 0)
    cols = jax.lax.broadcasted_iota(jnp.int32, (_RB, _N), 1)
    mask = (g_ref[...] > 0.99) | (rows == cols)
    e = jnp.where(mask, e, _NEG)
    m = jnp.max(e, axis=1, keepdims=True)
    p = jnp.exp(e - m)
    s = jnp.sum(p, axis=1, keepdims=True)
    C = p / s
    C_ref[...] = C
    o = x_ref[...] + jnp.dot(C, h_ref[...], preferred_element_type=jnp.float32)
    if relu:
        o = jnp.maximum(o, jnp.float32(0.0))
    o_ref[...] = o


def _att_layer(x, graph, h, q, kT, relu):
    grid = (_N // _RB,)
    import functools
    kern = functools.partial(_att_kernel, relu=relu)
    C, o = pl.pallas_call(
        kern,
        grid=grid,
        in_specs=[
            pl.BlockSpec((_RB, _D), lambda i: (i, 0)),
            pl.BlockSpec((_RB, _N), lambda i: (i, 0)),
            pl.BlockSpec((_RB, _A), lambda i: (i, 0)),
            pl.BlockSpec((_A, _N), lambda i: (0, 0)),
            pl.BlockSpec((_N, _D), lambda i: (0, 0)),
        ],
        out_specs=[
            pl.BlockSpec((_RB, _N), lambda i: (i, 0)),
            pl.BlockSpec((_RB, _D), lambda i: (i, 0)),
        ],
        out_shape=[
            jax.ShapeDtypeStruct((_N, _N), jnp.float32),
            jax.ShapeDtypeStruct((_N, _D), jnp.float32),
        ],
        compiler_params=pltpu.CompilerParams(
            vmem_limit_bytes=100 * 1024 * 1024,
        ),
    )(x, graph, q, kT, h)
    return C, o


def kernel(x, graph, W1, Wq1, Wk1, b1, W2, Wq2, Wk2, b2):
    h1, q1, kT1 = _project(x, W1, Wq1, Wk1, b1)
    C1, x1 = _att_layer(x, graph, h1, q1, kT1, relu=True)
    h2, q2, kT2 = _project(x1, W2, Wq2, Wk2, b2)
    C2, x2 = _att_layer(x1, graph, h2, q2, kT2, relu=False)
    return (x2, C1, C2)


# speedup vs baseline: 2.6164x; 1.0110x over previous
import math

import jax
import jax.numpy as jnp
import numpy as np
from jax.experimental import pallas as pl
from jax.experimental.pallas import tpu as pltpu

_N = 8192
_D = 64
_A = 32
_RB = 256
_PB = 1024

_INV_SQRT_A = np.float32(math.log2(math.e) / math.sqrt(_A))
_NEG = np.float32(-1e9)
_SLOPE = np.float32(0.2)


def _proj_kernel(x_ref, W_ref, Wq_ref, Wk_ref, b_ref, h_ref, q_ref, kT_ref):
    x = x_ref[...]
    h_ref[...] = (
        jnp.dot(x, W_ref[...], preferred_element_type=jnp.float32) + b_ref[...]
    )
    q_ref[...] = (
        jnp.dot(x, Wq_ref[...], preferred_element_type=jnp.float32) * _INV_SQRT_A
    )
    kT_ref[...] = jax.lax.dot_general(
        Wk_ref[...], x, (((0,), (1,)), ((), ())),
        preferred_element_type=jnp.float32,
    )


def _project(x, W, Wq, Wk, b):
    grid = (_N // _PB,)
    return pl.pallas_call(
        _proj_kernel,
        grid=grid,
        in_specs=[
            pl.BlockSpec((_PB, _D), lambda i: (i, 0)),
            pl.BlockSpec((_D, _D), lambda i: (0, 0)),
            pl.BlockSpec((_D, _A), lambda i: (0, 0)),
            pl.BlockSpec((_D, _A), lambda i: (0, 0)),
            pl.BlockSpec((1, _D), lambda i: (0, 0)),
        ],
        out_specs=[
            pl.BlockSpec((_PB, _D), lambda i: (i, 0)),
            pl.BlockSpec((_PB, _A), lambda i: (i, 0)),
            pl.BlockSpec((_A, _PB), lambda i: (0, i)),
        ],
        out_shape=[
            jax.ShapeDtypeStruct((_N, _D), jnp.float32),
            jax.ShapeDtypeStruct((_N, _A), jnp.float32),
            jax.ShapeDtypeStruct((_A, _N), jnp.float32),
        ],
    )(x, W, Wq, Wk, b.reshape(1, _D))


def _att_kernel(x_ref, g_ref, q_ref, kT_ref, h_ref, C_ref, o_ref, *, relu):
    i = pl.program_id(0)
    e = jnp.dot(q_ref[...], kT_ref[...], preferred_element_type=jnp.float32)
    e = jnp.maximum(e, e * _SLOPE)
    rows = jax.lax.broadcasted_iota(jnp.int32, (_RB, _N), 0)
    cols = jax.lax.broadcasted_iota(jnp.int32, (_RB, _N), 1)
    mask = (g_ref[...] > 0.99) | ((cols - rows) == i * _RB)
    e = jnp.where(mask, e, _NEG)
    m = jnp.max(e, axis=1, keepdims=True)
    p = jnp.exp2(e - m)
    s = jnp.sum(p, axis=1, keepdims=True)
    C = p * (jnp.float32(1.0) / s)
    C_ref[...] = C
    o = x_ref[...] + jnp.dot(C, h_ref[...], preferred_element_type=jnp.float32)
    if relu:
        o = jnp.maximum(o, jnp.float32(0.0))
    o_ref[...] = o


def _att_layer(x, graph, h, q, kT, relu):
    grid = (_N // _RB,)
    import functools
    kern = functools.partial(_att_kernel, relu=relu)
    C, o = pl.pallas_call(
        kern,
        grid=grid,
        in_specs=[
            pl.BlockSpec((_RB, _D), lambda i: (i, 0)),
            pl.BlockSpec((_RB, _N), lambda i: (i, 0)),
            pl.BlockSpec((_RB, _A), lambda i: (i, 0)),
            pl.BlockSpec((_A, _N), lambda i: (0, 0)),
            pl.BlockSpec((_N, _D), lambda i: (0, 0)),
        ],
        out_specs=[
            pl.BlockSpec((_RB, _N), lambda i: (i, 0)),
            pl.BlockSpec((_RB, _D), lambda i: (i, 0)),
        ],
        out_shape=[
            jax.ShapeDtypeStruct((_N, _N), jnp.float32),
            jax.ShapeDtypeStruct((_N, _D), jnp.float32),
        ],
        compiler_params=pltpu.CompilerParams(
            vmem_limit_bytes=100 * 1024 * 1024,
        ),
    )(x, graph, q, kT, h)
    return C, o


def kernel(x, graph, W1, Wq1, Wk1, b1, W2, Wq2, Wk2, b2):
    h1, q1, kT1 = _project(x, W1, Wq1, Wk1, b1)
    C1, x1 = _att_layer(x, graph, h1, q1, kT1, relu=True)
    h2, q2, kT2 = _project(x1, W2, Wq2, Wk2, b2)
    C2, x2 = _att_layer(x1, graph, h2, q2, kT2, relu=False)
    return (x2, C1, C2)
